# baseline (speedup 1.0000x reference)
import functools
import math

import jax
import jax.numpy as jnp
from jax import lax
from jax.experimental import pallas as pl
from jax.experimental.pallas import tpu as pltpu
from jax.experimental.pallas import tpu_sc as plsc

_B = 4096
_L = 64
_K = 256
_LANES = 128
_NTILES = 32
_RPT = _B // _NTILES

_C = -0.5 * math.log(2.0 * math.pi)
_NEG = -3.0e38


def _sc_body(zt_hbm, mt_hbm, lvt_hbm, lw_hbm, mx_hbm, s_hbm,
             z_v, m_t, a_t, p_t, lw_v, mx_v, s_v):
    wid = lax.axis_index("s") * 2 + lax.axis_index("c")
    pltpu.sync_copy(zt_hbm.at[wid], z_v)
    pltpu.sync_copy(mt_hbm, m_t)
    pltpu.sync_copy(lvt_hbm, p_t)
    pltpu.sync_copy(lw_hbm, lw_v)

    nkc = _K // 16
    nj = _RPT // 16

    def prep(l, carry):
        for kc in range(nkc):
            sl = pl.ds(16 * kc, 16)
            lw = lw_v[sl]
            lv = p_t[l, sl]
            a_t[l, sl] = (_C + lw) - 0.5 * lv
            p_t[l, sl] = 0.5 * jnp.exp(-lv)
        return carry

    lax.fori_loop(0, _L, prep, 0)

    def lbody(l, carry):
        zv = tuple(z_v[l, pl.ds(16 * j, 16)] for j in range(nj))

        def p1(kc, mxs):
            sl = pl.ds(16 * kc, 16)
            mv = m_t[l, sl]
            av = a_t[l, sl]
            pv = p_t[l, sl]
            mxs = list(mxs)
            for i in range(16):
                m, a, p = mv[i], av[i], pv[i]
                for j in range(nj):
                    d = zv[j] - m
                    mxs[j] = jnp.maximum(mxs[j], a - p * d * d)
            return tuple(mxs)

        mxs = lax.fori_loop(
            0, nkc, p1,
            tuple(jnp.full((16,), _NEG, jnp.float32) for _ in range(nj)))

        def p2(kc, ss):
            sl = pl.ds(16 * kc, 16)
            mv = m_t[l, sl]
            av = a_t[l, sl]
            pv = p_t[l, sl]
            ss = list(ss)
            for i in range(16):
                m, a, p = mv[i], av[i], pv[i]
                for j in range(nj):
                    d = zv[j] - m
                    ss[j] = ss[j] + jnp.exp((a - p * d * d) - mxs[j])
            return tuple(ss)

        ss = lax.fori_loop(
            0, nkc, p2, tuple(jnp.zeros((16,), jnp.float32) for _ in range(nj)))

        for j in range(nj):
            mx_v[l, pl.ds(16 * j, 16)] = mxs[j]
            s_v[l, pl.ds(16 * j, 16)] = ss[j]
        return carry

    lax.fori_loop(0, _L, lbody, 0)
    pltpu.sync_copy(mx_v, mx_hbm.at[wid])
    pltpu.sync_copy(s_v, s_hbm.at[wid])


_sc_mog = functools.partial(
    pl.kernel,
    mesh=plsc.VectorSubcoreMesh(core_axis_name="c", subcore_axis_name="s"),
    out_type=[
        jax.ShapeDtypeStruct((_NTILES, _L, _RPT), jnp.float32),
        jax.ShapeDtypeStruct((_NTILES, _L, _RPT), jnp.float32),
    ],
    scratch_types=[
        pltpu.VMEM((_L, _RPT), jnp.float32),
        pltpu.VMEM((_L, _K), jnp.float32),
        pltpu.VMEM((_L, _K), jnp.float32),
        pltpu.VMEM((_L, _K), jnp.float32),
        pltpu.VMEM((_K,), jnp.float32),
        pltpu.VMEM((_L, _RPT), jnp.float32),
        pltpu.VMEM((_L, _RPT), jnp.float32),
    ],
)(_sc_body)


def _fin_body(mx_ref, s_ref, o_ref):
    o_ref[...] = mx_ref[...] + jnp.log(s_ref[...])


def _finish(mx2, s2):
    rows = mx2.shape[0]
    return pl.pallas_call(
        _fin_body,
        out_shape=jax.ShapeDtypeStruct((rows, _LANES), jnp.float32),
    )(mx2, s2)


@jax.jit
def kernel(z, means, logvars, w):
    ws = w.reshape(_K)
    wmax = jnp.max(ws)
    logw = ws - (wmax + jnp.log(jnp.sum(jnp.exp(ws - wmax))))

    zt3 = z.reshape(_NTILES, _RPT, _L).transpose(0, 2, 1)
    mx3, s3 = _sc_mog(zt3, means.T, logvars.T, logw)
    out2 = _finish(mx3.reshape(-1, _LANES), s3.reshape(-1, _LANES))
    return (out2.reshape(_NTILES, _L, _RPT)
            .transpose(0, 2, 1)
            .reshape(_B, _L))


# speedup vs baseline: 8.3235x; 8.3235x over previous
import functools
import math

import jax
import jax.numpy as jnp
from jax import lax
from jax.experimental import pallas as pl
from jax.experimental.pallas import tpu as pltpu
from jax.experimental.pallas import tpu_sc as plsc

_B = 4096
_L = 64
_K = 256
_LANES = 128
_NTILES = 32

_SC_ROWS = 512
_RPT = _SC_ROWS // _NTILES
_TC_ROWS = _B - _SC_ROWS

_C = -0.5 * math.log(2.0 * math.pi)
_NEG = -3.0e38


def _sc_body(zt_hbm, mt_hbm, lvt_hbm, lw_hbm, mx_hbm, s_hbm,
             z_v, m_t, a_t, p_t, lw_v, mx_v, s_v):
    wid = lax.axis_index("s") * 2 + lax.axis_index("c")
    pltpu.sync_copy(zt_hbm.at[wid], z_v)
    pltpu.sync_copy(mt_hbm, m_t)
    pltpu.sync_copy(lvt_hbm, p_t)
    pltpu.sync_copy(lw_hbm, lw_v)

    nkc = _K // 16
    nj = _RPT // 16

    def prep(l, carry):
        for kc in range(nkc):
            sl = pl.ds(16 * kc, 16)
            lw = lw_v[sl]
            lv = p_t[l, sl]
            a_t[l, sl] = (_C + lw) - 0.5 * lv
            p_t[l, sl] = 0.5 * jnp.exp(-lv)
        return carry

    lax.fori_loop(0, _L, prep, 0)

    def lbody(l, carry):
        zv = tuple(z_v[l, pl.ds(16 * j, 16)] for j in range(nj))

        def p1(kc, mxs):
            sl = pl.ds(16 * kc, 16)
            mv = m_t[l, sl]
            av = a_t[l, sl]
            pv = p_t[l, sl]
            mxs = list(mxs)
            for ic in range(4):
                for j in range(nj):
                    ts = []
                    for i in range(4 * ic, 4 * ic + 4):
                        m, a, p = mv[i], av[i], pv[i]
                        d = zv[j] - m
                        ts.append(a - p * d * d)
                    t01 = jnp.maximum(ts[0], ts[1])
                    t23 = jnp.maximum(ts[2], ts[3])
                    mxs[j] = jnp.maximum(mxs[j], jnp.maximum(t01, t23))
            return tuple(mxs)

        mxs = lax.fori_loop(
            0, nkc, p1,
            tuple(jnp.full((16,), _NEG, jnp.float32) for _ in range(nj)))

        def p2(kc, ss):
            sl = pl.ds(16 * kc, 16)
            mv = m_t[l, sl]
            av = a_t[l, sl]
            pv = p_t[l, sl]
            ss = list(ss)
            for ic in range(4):
                for j in range(nj):
                    es = []
                    for i in range(4 * ic, 4 * ic + 4):
                        m, a, p = mv[i], av[i], pv[i]
                        d = zv[j] - m
                        es.append(jnp.exp((a - p * d * d) - mxs[j]))
                    e01 = es[0] + es[1]
                    e23 = es[2] + es[3]
                    ss[j] = ss[j] + (e01 + e23)
            return tuple(ss)

        ss = lax.fori_loop(
            0, nkc, p2, tuple(jnp.zeros((16,), jnp.float32) for _ in range(nj)))

        for j in range(nj):
            mx_v[l, pl.ds(16 * j, 16)] = mxs[j]
            s_v[l, pl.ds(16 * j, 16)] = ss[j]
        return carry

    lax.fori_loop(0, _L, lbody, 0)
    pltpu.sync_copy(mx_v, mx_hbm.at[wid])
    pltpu.sync_copy(s_v, s_hbm.at[wid])


_sc_mog = functools.partial(
    pl.kernel,
    mesh=plsc.VectorSubcoreMesh(core_axis_name="c", subcore_axis_name="s"),
    out_type=[
        jax.ShapeDtypeStruct((_NTILES, _L, _RPT), jnp.float32),
        jax.ShapeDtypeStruct((_NTILES, _L, _RPT), jnp.float32),
    ],
    scratch_types=[
        pltpu.VMEM((_L, _RPT), jnp.float32),
        pltpu.VMEM((_L, _K), jnp.float32),
        pltpu.VMEM((_L, _K), jnp.float32),
        pltpu.VMEM((_L, _K), jnp.float32),
        pltpu.VMEM((_K,), jnp.float32),
        pltpu.VMEM((_L, _RPT), jnp.float32),
        pltpu.VMEM((_L, _RPT), jnp.float32),
    ],
)(_sc_body)


_PACK = _LANES // _L
_TC_PROWS = _TC_ROWS // _PACK
_BLOCK_ROWS = 256
_TC_GRID = _TC_PROWS // _BLOCK_ROWS


def _tc_body(z_ref, m_ref, lv_ref, w_ref, o_ref, a_ref, p_ref):
    z = z_ref[...]
    lv = lv_ref[...]
    w = w_ref[...]
    wmax = jnp.max(w)
    logw = w - (wmax + jnp.log(jnp.sum(jnp.exp(w - wmax))))
    a_ref[...] = (_C + logw) - 0.5 * lv
    p_ref[...] = 0.5 * jnp.exp(-lv)

    def pass1(k, mx):
        d = z - m_ref[pl.ds(k, 1), :]
        t = a_ref[pl.ds(k, 1), :] - p_ref[pl.ds(k, 1), :] * d * d
        return jnp.maximum(mx, t)

    mx = lax.fori_loop(0, _K, pass1, jnp.full(z.shape, _NEG, jnp.float32),
                       unroll=8)

    def pass2(k, s):
        d = z - m_ref[pl.ds(k, 1), :]
        t = a_ref[pl.ds(k, 1), :] - p_ref[pl.ds(k, 1), :] * d * d
        return s + jnp.exp(t - mx)

    s = lax.fori_loop(0, _K, pass2, jnp.zeros(z.shape, jnp.float32),
                      unroll=8)
    o_ref[...] = mx + jnp.log(s)


def _tc_main(z2, m2, lv2, wc):
    return pl.pallas_call(
        _tc_body,
        grid=(_TC_GRID,),
        in_specs=[
            pl.BlockSpec((_BLOCK_ROWS, _LANES), lambda i: (i, 0)),
            pl.BlockSpec((_K, _LANES), lambda i: (0, 0)),
            pl.BlockSpec((_K, _LANES), lambda i: (0, 0)),
            pl.BlockSpec((_K, 1), lambda i: (0, 0)),
        ],
        out_specs=pl.BlockSpec((_BLOCK_ROWS, _LANES), lambda i: (i, 0)),
        out_shape=jax.ShapeDtypeStruct((_TC_PROWS, _LANES), jnp.float32),
        scratch_shapes=[
            pltpu.VMEM((_K, _LANES), jnp.float32),
            pltpu.VMEM((_K, _LANES), jnp.float32),
        ],
    )(z2, m2, lv2, wc)


def _fin_body(mx_ref, s_ref, o_ref):
    o_ref[...] = mx_ref[...] + jnp.log(s_ref[...])


def _finish(mx2, s2):
    rows = mx2.shape[0]
    return pl.pallas_call(
        _fin_body,
        out_shape=jax.ShapeDtypeStruct((rows, _LANES), jnp.float32),
    )(mx2, s2)


@jax.jit
def kernel(z, means, logvars, w):
    ws = w.reshape(_K)
    wmax = jnp.max(ws)
    logw = ws - (wmax + jnp.log(jnp.sum(jnp.exp(ws - wmax))))

    z_sc = z[:_SC_ROWS]
    zt3 = z_sc.reshape(_NTILES, _RPT, _L).transpose(0, 2, 1)
    mx3, s3 = _sc_mog(zt3, means.T, logvars.T, logw)

    z2 = z[_SC_ROWS:].reshape(_TC_PROWS, _LANES)
    m2 = jnp.concatenate([means, means], axis=1)
    lv2 = jnp.concatenate([logvars, logvars], axis=1)
    out_tc = _tc_main(z2, m2, lv2, w.reshape(_K, 1)).reshape(_TC_ROWS, _L)

    out_sc = (_finish(mx3.reshape(-1, _LANES), s3.reshape(-1, _LANES))
              .reshape(_NTILES, _L, _RPT)
              .transpose(0, 2, 1)
              .reshape(_SC_ROWS, _L))
    return jnp.concatenate([out_sc, out_tc], axis=0)


# speedup vs baseline: 8.4639x; 1.0169x over previous
import functools
import math

import jax
import jax.numpy as jnp
from jax import lax
from jax.experimental import pallas as pl
from jax.experimental.pallas import tpu as pltpu
from jax.experimental.pallas import tpu_sc as plsc

_B = 4096
_L = 64
_K = 256
_LANES = 128
_NTILES = 32

_SC_ROWS = 512
_RPT = _SC_ROWS // _NTILES
_TC_ROWS = _B - _SC_ROWS

_C = -0.5 * math.log(2.0 * math.pi)
_NEG = -3.0e38


def _sc_body(zt_hbm, mt_hbm, lvt_hbm, lw_hbm, mx_hbm, s_hbm,
             z_v, m_t, a_t, p_t, lw_v, mx_v, s_v):
    wid = lax.axis_index("s") * 2 + lax.axis_index("c")
    pltpu.sync_copy(zt_hbm.at[wid], z_v)
    pltpu.sync_copy(mt_hbm, m_t)
    pltpu.sync_copy(lvt_hbm, p_t)
    pltpu.sync_copy(lw_hbm, lw_v)

    nkc = _K // 16
    nj = _RPT // 16

    def prep(l, carry):
        for kc in range(nkc):
            sl = pl.ds(16 * kc, 16)
            lw = lw_v[sl]
            lv = p_t[l, sl]
            a_t[l, sl] = (_C + lw) - 0.5 * lv
            p_t[l, sl] = 0.5 * jnp.exp(-lv)
        return carry

    lax.fori_loop(0, _L, prep, 0)

    def lbody(l, carry):
        zv = tuple(z_v[l, pl.ds(16 * j, 16)] for j in range(nj))

        def p1(kc, mxs):
            sl = pl.ds(16 * kc, 16)
            mv = m_t[l, sl]
            av = a_t[l, sl]
            pv = p_t[l, sl]
            mxs = list(mxs)
            for ic in range(4):
                for j in range(nj):
                    ts = []
                    for i in range(4 * ic, 4 * ic + 4):
                        m, a, p = mv[i], av[i], pv[i]
                        d = zv[j] - m
                        ts.append(a - p * d * d)
                    t01 = jnp.maximum(ts[0], ts[1])
                    t23 = jnp.maximum(ts[2], ts[3])
                    mxs[j] = jnp.maximum(mxs[j], jnp.maximum(t01, t23))
            return tuple(mxs)

        mxs = lax.fori_loop(
            0, nkc, p1,
            tuple(jnp.full((16,), _NEG, jnp.float32) for _ in range(nj)))

        def p2(kc, ss):
            sl = pl.ds(16 * kc, 16)
            mv = m_t[l, sl]
            av = a_t[l, sl]
            pv = p_t[l, sl]
            ss = list(ss)
            for ic in range(4):
                for j in range(nj):
                    es = []
                    for i in range(4 * ic, 4 * ic + 4):
                        m, a, p = mv[i], av[i], pv[i]
                        d = zv[j] - m
                        es.append(jnp.exp((a - p * d * d) - mxs[j]))
                    e01 = es[0] + es[1]
                    e23 = es[2] + es[3]
                    ss[j] = ss[j] + (e01 + e23)
            return tuple(ss)

        ss = lax.fori_loop(
            0, nkc, p2, tuple(jnp.zeros((16,), jnp.float32) for _ in range(nj)))

        for j in range(nj):
            mx_v[l, pl.ds(16 * j, 16)] = mxs[j]
            s_v[l, pl.ds(16 * j, 16)] = ss[j]
        return carry

    lax.fori_loop(0, _L, lbody, 0)
    pltpu.sync_copy(mx_v, mx_hbm.at[wid])
    pltpu.sync_copy(s_v, s_hbm.at[wid])


_sc_mog = functools.partial(
    pl.kernel,
    mesh=plsc.VectorSubcoreMesh(core_axis_name="c", subcore_axis_name="s"),
    out_type=[
        jax.ShapeDtypeStruct((_NTILES, _L, _RPT), jnp.float32),
        jax.ShapeDtypeStruct((_NTILES, _L, _RPT), jnp.float32),
    ],
    scratch_types=[
        pltpu.VMEM((_L, _RPT), jnp.float32),
        pltpu.VMEM((_L, _K), jnp.float32),
        pltpu.VMEM((_L, _K), jnp.float32),
        pltpu.VMEM((_L, _K), jnp.float32),
        pltpu.VMEM((_K,), jnp.float32),
        pltpu.VMEM((_L, _RPT), jnp.float32),
        pltpu.VMEM((_L, _RPT), jnp.float32),
    ],
)(_sc_body)


_PACK = _LANES // _L
_TC_PROWS = _TC_ROWS // _PACK
_BLOCK_ROWS = 256
_TC_GRID = _TC_PROWS // _BLOCK_ROWS


def _tc_body(z_ref, m_ref, lv_ref, w_ref, o_ref, a_ref, p_ref):
    z = z_ref[...]
    lv = lv_ref[...]
    w = w_ref[...]
    wmax = jnp.max(w)
    logw = w - (wmax + jnp.log(jnp.sum(jnp.exp(w - wmax))))
    a_ref[...] = (_C + logw) - 0.5 * lv
    p_ref[...] = 0.5 * jnp.exp(-lv)

    def pass1(k, mx):
        d = z - m_ref[pl.ds(k, 1), :]
        t = a_ref[pl.ds(k, 1), :] - p_ref[pl.ds(k, 1), :] * d * d
        return jnp.maximum(mx, t)

    mx = lax.fori_loop(0, _K, pass1, jnp.full(z.shape, _NEG, jnp.float32),
                       unroll=16)

    def pass2(k, s):
        d = z - m_ref[pl.ds(k, 1), :]
        t = a_ref[pl.ds(k, 1), :] - p_ref[pl.ds(k, 1), :] * d * d
        return s + jnp.exp(t - mx)

    s = lax.fori_loop(0, _K, pass2, jnp.zeros(z.shape, jnp.float32),
                      unroll=16)
    o_ref[...] = mx + jnp.log(s)


def _tc_main(z2, m2, lv2, wc):
    return pl.pallas_call(
        _tc_body,
        grid=(_TC_GRID,),
        in_specs=[
            pl.BlockSpec((_BLOCK_ROWS, _LANES), lambda i: (i, 0)),
            pl.BlockSpec((_K, _LANES), lambda i: (0, 0)),
            pl.BlockSpec((_K, _LANES), lambda i: (0, 0)),
            pl.BlockSpec((_K, 1), lambda i: (0, 0)),
        ],
        out_specs=pl.BlockSpec((_BLOCK_ROWS, _LANES), lambda i: (i, 0)),
        out_shape=jax.ShapeDtypeStruct((_TC_PROWS, _LANES), jnp.float32),
        scratch_shapes=[
            pltpu.VMEM((_K, _LANES), jnp.float32),
            pltpu.VMEM((_K, _LANES), jnp.float32),
        ],
    )(z2, m2, lv2, wc)


def _fin_body(mx_ref, s_ref, o_ref):
    o_ref[...] = mx_ref[...] + jnp.log(s_ref[...])


def _finish(mx2, s2):
    rows = mx2.shape[0]
    return pl.pallas_call(
        _fin_body,
        out_shape=jax.ShapeDtypeStruct((rows, _LANES), jnp.float32),
    )(mx2, s2)


@jax.jit
def kernel(z, means, logvars, w):
    ws = w.reshape(_K)
    wmax = jnp.max(ws)
    logw = ws - (wmax + jnp.log(jnp.sum(jnp.exp(ws - wmax))))

    z_sc = z[:_SC_ROWS]
    zt3 = z_sc.reshape(_NTILES, _RPT, _L).transpose(0, 2, 1)
    mx3, s3 = _sc_mog(zt3, means.T, logvars.T, logw)

    z2 = z[_SC_ROWS:].reshape(_TC_PROWS, _LANES)
    m2 = jnp.concatenate([means, means], axis=1)
    lv2 = jnp.concatenate([logvars, logvars], axis=1)
    out_tc = _tc_main(z2, m2, lv2, w.reshape(_K, 1)).reshape(_TC_ROWS, _L)

    out_sc = (_finish(mx3.reshape(-1, _LANES), s3.reshape(-1, _LANES))
              .reshape(_NTILES, _L, _RPT)
              .transpose(0, 2, 1)
              .reshape(_SC_ROWS, _L))
    return jnp.concatenate([out_sc, out_tc], axis=0)
